# Initial kernel scaffold; baseline (speedup 1.0000x reference)
import functools

import jax
import jax.numpy as jnp
from jax import lax
from jax.experimental import pallas as pl
from jax.experimental.pallas import tpu as pltpu
from jax.experimental.pallas import tpu_sc as plsc

N = 10000
E = 320000
D = 128
NSUB = 16
NCORE = 2
CHUNK = 125
NCHUNK = E // (NSUB * CHUNK)
ROWS_PER_TILE = N // NSUB


def _sc_propagate(src, gidx, sidx):
    mesh = plsc.VectorSubcoreMesh(core_axis_name="c", subcore_axis_name="s")

    @functools.partial(
        pl.kernel,
        mesh=mesh,
        out_type=jax.ShapeDtypeStruct((NCORE, N, D), jnp.float32),
        scratch_types=[
            pltpu.VMEM((NCHUNK, CHUNK), jnp.int32),
            pltpu.VMEM((NCHUNK, CHUNK), jnp.int32),
            pltpu.VMEM((CHUNK, D), jnp.float32),
            pltpu.VMEM_SHARED((N, D), jnp.float32),
            pltpu.SemaphoreType.DMA,
        ],
    )
    def k(src_hbm, gidx_hbm, sidx_hbm, out_hbm, gidx_v, sidx_v, rows_v, acc, sem):
        cid = lax.axis_index("c")
        sid = lax.axis_index("s")

        pltpu.sync_copy(gidx_hbm.at[cid, sid], gidx_v)
        pltpu.sync_copy(sidx_hbm.at[cid, sid], sidx_v)

        zeros16 = jnp.zeros((16,), jnp.float32)

        def zero_row(i, carry):
            for kk in range(D // 16):
                rows_v[i, pl.ds(kk * 16, 16)] = zeros16
            return carry

        lax.fori_loop(0, CHUNK, zero_row, 0)
        for b in range(ROWS_PER_TILE // CHUNK):
            pltpu.sync_copy(
                rows_v, acc.at[pl.ds(sid * ROWS_PER_TILE + b * CHUNK, CHUNK)])
        plsc.subcore_barrier()

        def body(j, carry):
            pltpu.async_copy(src_hbm.at[gidx_v.at[j]], rows_v, sem).wait()
            pltpu.sync_copy(rows_v, acc.at[sidx_v.at[j]], add=True)
            return carry

        lax.fori_loop(0, NCHUNK, body, 0)
        plsc.subcore_barrier()

        pltpu.sync_copy(
            acc.at[pl.ds(sid * ROWS_PER_TILE, ROWS_PER_TILE)],
            out_hbm.at[cid, pl.ds(sid * ROWS_PER_TILE, ROWS_PER_TILE)])

    return k(src, gidx, sidx)


_R = 1000


def _tc_layer1(x, agg, A, B, WmT, biases):
    def body(x_ref, agg_ref, A_ref, B_ref, WmT_ref, b_ref, o_ref):
        xb = x_ref[...]
        for m in range(2):
            hd = jnp.dot(agg_ref[m], A_ref[m], preferred_element_type=jnp.float32)
            hd = hd + jnp.dot(xb, B_ref[m], preferred_element_type=jnp.float32)
            hd = hd + b_ref[m][None, :] + xb
            h = jnp.maximum(hd, 0.0)
            o_ref[m] = (jnp.dot(h, WmT_ref[m], preferred_element_type=jnp.float32)
                        + b_ref[2 + m][None, :])

    return pl.pallas_call(
        body,
        grid=(N // _R,),
        in_specs=[
            pl.BlockSpec((_R, D), lambda i: (i, 0)),
            pl.BlockSpec((2, _R, D), lambda i: (0, i, 0)),
            pl.BlockSpec((2, D, D), lambda i: (0, 0, 0)),
            pl.BlockSpec((2, D, D), lambda i: (0, 0, 0)),
            pl.BlockSpec((2, D, D), lambda i: (0, 0, 0)),
            pl.BlockSpec((8, D), lambda i: (0, 0)),
        ],
        out_specs=pl.BlockSpec((2, _R, D), lambda i: (0, i, 0)),
        out_shape=jax.ShapeDtypeStruct((2, N, D), jnp.float32),
    )(x, agg, A, B, WmT, biases)


def _tc_final(h1, agg, A, B, WmT, WoutT, Wqkvo, Wp1T, biases, biases2):
    H, HD = 4, 32
    scale = 1.0 / (HD ** 0.5)

    def body(h1_ref, agg_ref, A_ref, B_ref, WmT_ref, WoutT_ref, Wq_ref,
             Wp1_ref, b_ref, b2_ref, o_ref):
        u = []
        for m in range(2):
            hm = h1_ref[m]
            hd = jnp.dot(agg_ref[m], A_ref[m], preferred_element_type=jnp.float32)
            hd = hd + jnp.dot(hm, B_ref[m], preferred_element_type=jnp.float32)
            hd = hd + b_ref[m][None, :] + hm
            h = jnp.maximum(hd, 0.0)
            h2 = (jnp.dot(h, WmT_ref[m], preferred_element_type=jnp.float32)
                  + b_ref[2 + m][None, :])
            u.append(jnp.dot(h2, WoutT_ref[m], preferred_element_type=jnp.float32)
                     + b_ref[4 + m][None, :])

        q = [jnp.dot(u[m], Wq_ref[0], preferred_element_type=jnp.float32)
             + b2_ref[0][None, :] for m in range(2)]
        kk = [jnp.dot(u[m], Wq_ref[1], preferred_element_type=jnp.float32)
              + b2_ref[1][None, :] for m in range(2)]
        v = [jnp.dot(u[m], Wq_ref[2], preferred_element_type=jnp.float32)
             + b2_ref[2][None, :] for m in range(2)]

        def head_scores(a, b):
            p = a * b
            return [jnp.sum(p[:, h * HD:(h + 1) * HD], axis=1, keepdims=True) * scale
                    for h in range(H)]

        s00 = head_scores(q[0], kk[0])
        s01 = head_scores(q[0], kk[1])
        s10 = head_scores(q[1], kk[0])
        s11 = head_scores(q[1], kk[1])

        att = []
        for si0, si1 in ((s00, s01), (s10, s11)):
            chunks = []
            for h in range(H):
                mx = jnp.maximum(si0[h], si1[h])
                e0 = jnp.exp(si0[h] - mx)
                e1 = jnp.exp(si1[h] - mx)
                inv = 1.0 / (e0 + e1)
                a0 = e0 * inv
                a1 = e1 * inv
                chunks.append(a0 * v[0][:, h * HD:(h + 1) * HD]
                              + a1 * v[1][:, h * HD:(h + 1) * HD])
            ctx = jnp.concatenate(chunks, axis=1)
            att.append(jnp.dot(ctx, Wq_ref[3], preferred_element_type=jnp.float32)
                       + b2_ref[3][None, :])

        pooled = 0.5 * (att[0] + att[1])
        z = jnp.maximum(
            jnp.dot(pooled, Wp1_ref[...], preferred_element_type=jnp.float32)
            + b2_ref[4][None, :], 0.0)
        o_ref[...] = jnp.sum(z * b2_ref[5][None, :], axis=1) + b2_ref[6, 0]

    return pl.pallas_call(
        body,
        grid=(N // _R,),
        in_specs=[
            pl.BlockSpec((2, _R, D), lambda i: (0, i, 0)),
            pl.BlockSpec((2, _R, D), lambda i: (0, i, 0)),
            pl.BlockSpec((2, D, D), lambda i: (0, 0, 0)),
            pl.BlockSpec((2, D, D), lambda i: (0, 0, 0)),
            pl.BlockSpec((2, D, D), lambda i: (0, 0, 0)),
            pl.BlockSpec((2, D, D), lambda i: (0, 0, 0)),
            pl.BlockSpec((4, D, D), lambda i: (0, 0, 0)),
            pl.BlockSpec((D, D), lambda i: (0, 0)),
            pl.BlockSpec((8, D), lambda i: (0, 0)),
            pl.BlockSpec((8, D), lambda i: (0, 0)),
        ],
        out_specs=pl.BlockSpec((_R,), lambda i: (i,)),
        out_shape=jax.ShapeDtypeStruct((N,), jnp.float32),
    )(h1, agg, A, B, WmT, WoutT, Wqkvo, Wp1T, biases, biases2)


def kernel(x, ei0, ei1, ei2, ei3, Wl, bl, W0, b0, W1, b1, Wm, bm, Wout, bout,
           Wq, bq, Wk, bk, Wv, bv, Wo, bo, Wp1, bp1, Wp2, bp2, mp_weights):
    def shape_idx(a):
        return a.reshape(NSUB, NCHUNK, CHUNK)

    g1 = jnp.stack([shape_idx(ei1[1]), shape_idx(ei3[1])])
    s1 = jnp.stack([shape_idx(ei1[0]), shape_idx(ei3[0])])
    g0 = jnp.stack([shape_idx(ei0[1]), shape_idx(ei2[1] + N)])
    s0 = jnp.stack([shape_idx(ei0[0]), shape_idx(ei2[0])])

    WlT = jnp.swapaxes(Wl, -1, -2)
    BT = jnp.swapaxes(W0 + W1, -1, -2)
    bsum = bl + b0 + b1
    WmT = jnp.swapaxes(Wm, -1, -2)
    WoutT = jnp.swapaxes(Wout, -1, -2) * mp_weights[:, None, None]
    boutw = bout * mp_weights[:, None]

    pad = jnp.zeros((4, D), jnp.float32)
    biases1 = jnp.concatenate([bsum[:, 1], bm[:, 1], pad], axis=0)
    biases0 = jnp.concatenate([bsum[:, 0], bm[:, 0], boutw,
                               jnp.zeros((2, D), jnp.float32)], axis=0)
    Wqkvo = jnp.stack([Wq.T, Wk.T, Wv.T, Wo.T])
    biases2 = jnp.stack([bq, bk, bv, bo, bp1, Wp2[0],
                         jnp.full((D,), bp2[0], jnp.float32),
                         jnp.zeros((D,), jnp.float32)])

    agg1 = _sc_propagate(x, g1, s1)
    h1 = _tc_layer1(x, agg1, WlT[:, 1], BT[:, 1], WmT[:, 1], biases1)

    agg0 = _sc_propagate(h1.reshape(2 * N, D), g0, s0)
    out = _tc_final(h1, agg0, WlT[:, 0], BT[:, 0], WmT[:, 0], WoutT,
                    Wqkvo, Wp1.T, biases0, biases2)
    return out


# speedup vs baseline: 5.8133x; 5.8133x over previous
import functools

import jax
import jax.numpy as jnp
from jax import lax
from jax.experimental import pallas as pl
from jax.experimental.pallas import tpu as pltpu
from jax.experimental.pallas import tpu_sc as plsc

N = 10000
E = 320000
D = 128
NSUB = 16
NCORE = 2
CHUNK = 125
NCHUNK = E // (NSUB * CHUNK)
RPT = 624
ZC = 104


def _sc_propagate(src, gidx, sidx):
    mesh = plsc.VectorSubcoreMesh(core_axis_name="c", subcore_axis_name="s")

    @functools.partial(
        pl.kernel,
        mesh=mesh,
        out_type=jax.ShapeDtypeStruct((NCORE, N, D), jnp.float32),
        scratch_types=[
            pltpu.VMEM((NCHUNK // 2, CHUNK), jnp.int32),
            pltpu.VMEM((NCHUNK // 2, CHUNK), jnp.int32),
            pltpu.VMEM((CHUNK, D), jnp.float32),
            pltpu.VMEM_SHARED((N, D), jnp.float32),
            pltpu.SemaphoreType.DMA,
        ],
    )
    def k(src_hbm, gidx_hbm, sidx_hbm, out_hbm, gidx_v, sidx_v, rows_v, acc, sem):
        cid = lax.axis_index("c")
        sid = lax.axis_index("s")

        zeros16 = jnp.zeros((16,), jnp.float32)

        def zero_row(i, carry):
            for kk in range(D // 16):
                rows_v[i, pl.ds(kk * 16, 16)] = zeros16
            return carry

        lax.fori_loop(0, CHUNK, zero_row, 0)
        for b in range(RPT // ZC):
            pltpu.sync_copy(
                rows_v.at[pl.ds(0, ZC)], acc.at[pl.ds(sid * RPT + b * ZC, ZC)])

        @pl.when(sid == NSUB - 1)
        def _zero_tail():
            pltpu.sync_copy(rows_v.at[pl.ds(0, 16)],
                            acc.at[pl.ds(NSUB * RPT, 16)])

        plsc.subcore_barrier()

        def body(j, carry):
            pltpu.async_copy(src_hbm.at[gidx_v.at[j]], rows_v, sem).wait()
            pltpu.sync_copy(rows_v, acc.at[sidx_v.at[j]], add=True)
            return carry

        for half in range(2):
            pltpu.sync_copy(
                gidx_hbm.at[cid, sid, pl.ds(half * (NCHUNK // 2), NCHUNK // 2)],
                gidx_v)
            pltpu.sync_copy(
                sidx_hbm.at[cid, sid, pl.ds(half * (NCHUNK // 2), NCHUNK // 2)],
                sidx_v)
            lax.fori_loop(0, NCHUNK // 2, body, 0)
        plsc.subcore_barrier()

        pltpu.sync_copy(
            acc.at[pl.ds(sid * RPT, RPT)],
            out_hbm.at[cid, pl.ds(sid * RPT, RPT)])

        @pl.when(sid == NSUB - 1)
        def _copy_tail():
            pltpu.sync_copy(acc.at[pl.ds(NSUB * RPT, 16)],
                            out_hbm.at[cid, pl.ds(NSUB * RPT, 16)])

    return k(src, gidx, sidx)


_R = 1000


def _tc_layer1(x, agg, A, B, WmT, biases):
    def body(x_ref, agg_ref, A_ref, B_ref, WmT_ref, b_ref, o_ref):
        xb = x_ref[...]
        for m in range(2):
            hd = jnp.dot(agg_ref[m], A_ref[m], preferred_element_type=jnp.float32, precision=lax.Precision.HIGHEST)
            hd = hd + jnp.dot(xb, B_ref[m], preferred_element_type=jnp.float32, precision=lax.Precision.HIGHEST)
            hd = hd + b_ref[m][None, :] + xb
            h = jnp.maximum(hd, 0.0)
            o_ref[m] = (jnp.dot(h, WmT_ref[m], preferred_element_type=jnp.float32, precision=lax.Precision.HIGHEST)
                        + b_ref[2 + m][None, :])

    return pl.pallas_call(
        body,
        grid=(N // _R,),
        in_specs=[
            pl.BlockSpec((_R, D), lambda i: (i, 0)),
            pl.BlockSpec((2, _R, D), lambda i: (0, i, 0)),
            pl.BlockSpec((2, D, D), lambda i: (0, 0, 0)),
            pl.BlockSpec((2, D, D), lambda i: (0, 0, 0)),
            pl.BlockSpec((2, D, D), lambda i: (0, 0, 0)),
            pl.BlockSpec((8, D), lambda i: (0, 0)),
        ],
        out_specs=pl.BlockSpec((2, _R, D), lambda i: (0, i, 0)),
        out_shape=jax.ShapeDtypeStruct((2, N, D), jnp.float32),
    )(x, agg, A, B, WmT, biases)


def _tc_final(h1, agg, A, B, WmT, WoutT, Wqkvo, Wp1T, biases, biases2):
    H, HD = 4, 32
    scale = 1.0 / (HD ** 0.5)

    def body(h1_ref, agg_ref, A_ref, B_ref, WmT_ref, WoutT_ref, Wq_ref,
             Wp1_ref, b_ref, b2_ref, o_ref):
        u = []
        for m in range(2):
            hm = h1_ref[m]
            hd = jnp.dot(agg_ref[m], A_ref[m], preferred_element_type=jnp.float32, precision=lax.Precision.HIGHEST)
            hd = hd + jnp.dot(hm, B_ref[m], preferred_element_type=jnp.float32, precision=lax.Precision.HIGHEST)
            hd = hd + b_ref[m][None, :] + hm
            h = jnp.maximum(hd, 0.0)
            h2 = (jnp.dot(h, WmT_ref[m], preferred_element_type=jnp.float32, precision=lax.Precision.HIGHEST)
                  + b_ref[2 + m][None, :])
            u.append(jnp.dot(h2, WoutT_ref[m], preferred_element_type=jnp.float32, precision=lax.Precision.HIGHEST)
                     + b_ref[4 + m][None, :])

        q = [jnp.dot(u[m], Wq_ref[0], preferred_element_type=jnp.float32, precision=lax.Precision.HIGHEST)
             + b2_ref[0][None, :] for m in range(2)]
        kk = [jnp.dot(u[m], Wq_ref[1], preferred_element_type=jnp.float32, precision=lax.Precision.HIGHEST)
              + b2_ref[1][None, :] for m in range(2)]
        v = [jnp.dot(u[m], Wq_ref[2], preferred_element_type=jnp.float32, precision=lax.Precision.HIGHEST)
             + b2_ref[2][None, :] for m in range(2)]

        def head_scores(a, b):
            p = a * b
            return [jnp.sum(p[:, h * HD:(h + 1) * HD], axis=1, keepdims=True) * scale
                    for h in range(H)]

        s00 = head_scores(q[0], kk[0])
        s01 = head_scores(q[0], kk[1])
        s10 = head_scores(q[1], kk[0])
        s11 = head_scores(q[1], kk[1])

        att = []
        for si0, si1 in ((s00, s01), (s10, s11)):
            chunks = []
            for h in range(H):
                mx = jnp.maximum(si0[h], si1[h])
                e0 = jnp.exp(si0[h] - mx)
                e1 = jnp.exp(si1[h] - mx)
                inv = 1.0 / (e0 + e1)
                a0 = e0 * inv
                a1 = e1 * inv
                chunks.append(a0 * v[0][:, h * HD:(h + 1) * HD]
                              + a1 * v[1][:, h * HD:(h + 1) * HD])
            ctx = jnp.concatenate(chunks, axis=1)
            att.append(jnp.dot(ctx, Wq_ref[3], preferred_element_type=jnp.float32, precision=lax.Precision.HIGHEST)
                       + b2_ref[3][None, :])

        pooled = 0.5 * (att[0] + att[1])
        z = jnp.maximum(
            jnp.dot(pooled, Wp1_ref[...], preferred_element_type=jnp.float32, precision=lax.Precision.HIGHEST)
            + b2_ref[4][None, :], 0.0)
        o_ref[0, 0] = jnp.sum(z * b2_ref[5][None, :], axis=1) + b2_ref[6, 0]

    return pl.pallas_call(
        body,
        grid=(N // _R,),
        in_specs=[
            pl.BlockSpec((2, _R, D), lambda i: (0, i, 0)),
            pl.BlockSpec((2, _R, D), lambda i: (0, i, 0)),
            pl.BlockSpec((2, D, D), lambda i: (0, 0, 0)),
            pl.BlockSpec((2, D, D), lambda i: (0, 0, 0)),
            pl.BlockSpec((2, D, D), lambda i: (0, 0, 0)),
            pl.BlockSpec((2, D, D), lambda i: (0, 0, 0)),
            pl.BlockSpec((4, D, D), lambda i: (0, 0, 0)),
            pl.BlockSpec((D, D), lambda i: (0, 0)),
            pl.BlockSpec((8, D), lambda i: (0, 0)),
            pl.BlockSpec((8, D), lambda i: (0, 0)),
        ],
        out_specs=pl.BlockSpec((1, 1, _R), lambda i: (i, 0, 0)),
        out_shape=jax.ShapeDtypeStruct((N // _R, 1, _R), jnp.float32),
    )(h1, agg, A, B, WmT, WoutT, Wqkvo, Wp1T, biases, biases2)


def kernel(x, ei0, ei1, ei2, ei3, Wl, bl, W0, b0, W1, b1, Wm, bm, Wout, bout,
           Wq, bq, Wk, bk, Wv, bv, Wo, bo, Wp1, bp1, Wp2, bp2, mp_weights):
    def shape_idx(a):
        return a.reshape(NSUB, NCHUNK, CHUNK)

    g1 = jnp.stack([shape_idx(ei1[1]), shape_idx(ei3[1])])
    s1 = jnp.stack([shape_idx(ei1[0]), shape_idx(ei3[0])])
    g0 = jnp.stack([shape_idx(ei0[1]), shape_idx(ei2[1] + N)])
    s0 = jnp.stack([shape_idx(ei0[0]), shape_idx(ei2[0])])

    WlT = jnp.swapaxes(Wl, -1, -2)
    BT = jnp.swapaxes(W0 + W1, -1, -2)
    bsum = bl + b0 + b1
    WmT = jnp.swapaxes(Wm, -1, -2)
    WoutT = jnp.swapaxes(Wout, -1, -2) * mp_weights[:, None, None]
    boutw = bout * mp_weights[:, None]

    pad = jnp.zeros((4, D), jnp.float32)
    biases1 = jnp.concatenate([bsum[:, 1], bm[:, 1], pad], axis=0)
    biases0 = jnp.concatenate([bsum[:, 0], bm[:, 0], boutw,
                               jnp.zeros((2, D), jnp.float32)], axis=0)
    Wqkvo = jnp.stack([Wq.T, Wk.T, Wv.T, Wo.T])
    biases2 = jnp.stack([bq, bk, bv, bo, bp1, Wp2[0],
                         jnp.full((D,), bp2[0], jnp.float32),
                         jnp.zeros((D,), jnp.float32)])

    agg1 = _sc_propagate(x, g1, s1)
    h1 = _tc_layer1(x, agg1, WlT[:, 1], BT[:, 1], WmT[:, 1], biases1)

    agg0 = _sc_propagate(h1.reshape(2 * N, D), g0, s0)
    out = _tc_final(h1, agg0, WlT[:, 0], BT[:, 0], WmT[:, 0], WoutT,
                    Wqkvo, Wp1.T, biases0, biases2)
    return out.reshape(N)


# speedup vs baseline: 7.7187x; 1.3278x over previous
import functools

import jax
import jax.numpy as jnp
from jax import lax
from jax.experimental import pallas as pl
from jax.experimental.pallas import tpu as pltpu
from jax.experimental.pallas import tpu_sc as plsc

N = 10000
E = 320000
D = 128
NSUB = 16
NCORE = 2
CHUNK = 125
NCHUNK = E // (NSUB * CHUNK)
RPT = 624
ZC = 104


def _sc_propagate(src, gidx, sidx):
    mesh = plsc.VectorSubcoreMesh(core_axis_name="c", subcore_axis_name="s")

    @functools.partial(
        pl.kernel,
        mesh=mesh,
        out_type=jax.ShapeDtypeStruct((NCORE, N, D), jnp.float32),
        scratch_types=[
            pltpu.VMEM((NCHUNK // 4, CHUNK), jnp.int32),
            pltpu.VMEM((NCHUNK // 4, CHUNK), jnp.int32),
            pltpu.VMEM((CHUNK, D), jnp.float32),
            pltpu.VMEM((CHUNK, D), jnp.float32),
            pltpu.VMEM_SHARED((N, D), jnp.float32),
            pltpu.SemaphoreType.DMA,
            pltpu.SemaphoreType.DMA,
        ],
    )
    def k(src_hbm, gidx_hbm, sidx_hbm, out_hbm, gidx_v, sidx_v, rows0, rows1,
          acc, sem0, sem1):
        rows_v = rows0
        cid = lax.axis_index("c")
        sid = lax.axis_index("s")

        zeros16 = jnp.zeros((16,), jnp.float32)

        def zero_row(i, carry):
            for kk in range(D // 16):
                rows_v[i, pl.ds(kk * 16, 16)] = zeros16
            return carry

        lax.fori_loop(0, CHUNK, zero_row, 0)
        for b in range(RPT // ZC):
            pltpu.sync_copy(
                rows_v.at[pl.ds(0, ZC)], acc.at[pl.ds(sid * RPT + b * ZC, ZC)])

        @pl.when(sid == NSUB - 1)
        def _zero_tail():
            pltpu.sync_copy(rows_v.at[pl.ds(0, 16)],
                            acc.at[pl.ds(NSUB * RPT, 16)])

        plsc.subcore_barrier()

        Q = NCHUNK // 4

        def step(j, rows, sem):
            pltpu.make_async_copy(src_hbm.at[gidx_v.at[j]], rows, sem).wait()
            pltpu.sync_copy(rows, acc.at[sidx_v.at[j]], add=True)

        def body(p, carry):
            j = 2 * p
            step(j, rows0, sem0)
            pltpu.async_copy(src_hbm.at[gidx_v.at[j + 2]], rows0, sem0)
            step(j + 1, rows1, sem1)
            pltpu.async_copy(src_hbm.at[gidx_v.at[j + 3]], rows1, sem1)
            return carry

        for quarter in range(4):
            pltpu.sync_copy(gidx_hbm.at[cid, sid, pl.ds(quarter * Q, Q)],
                            gidx_v)
            pltpu.sync_copy(sidx_hbm.at[cid, sid, pl.ds(quarter * Q, Q)],
                            sidx_v)
            pltpu.async_copy(src_hbm.at[gidx_v.at[0]], rows0, sem0)
            pltpu.async_copy(src_hbm.at[gidx_v.at[1]], rows1, sem1)
            lax.fori_loop(0, Q // 2 - 1, body, 0)
            step(Q - 2, rows0, sem0)
            step(Q - 1, rows1, sem1)
        plsc.subcore_barrier()

        pltpu.sync_copy(
            acc.at[pl.ds(sid * RPT, RPT)],
            out_hbm.at[cid, pl.ds(sid * RPT, RPT)])

        @pl.when(sid == NSUB - 1)
        def _copy_tail():
            pltpu.sync_copy(acc.at[pl.ds(NSUB * RPT, 16)],
                            out_hbm.at[cid, pl.ds(NSUB * RPT, 16)])

    return k(src, gidx, sidx)


_R = 1000


def _tc_layer1(x, agg, A, B, WmT, biases):
    def body(x_ref, agg_ref, A_ref, B_ref, WmT_ref, b_ref, o_ref):
        xb = x_ref[...]
        for m in range(2):
            hd = jnp.dot(agg_ref[m], A_ref[m], preferred_element_type=jnp.float32, precision=lax.Precision.HIGHEST)
            hd = hd + jnp.dot(xb, B_ref[m], preferred_element_type=jnp.float32, precision=lax.Precision.HIGHEST)
            hd = hd + b_ref[m][None, :] + xb
            h = jnp.maximum(hd, 0.0)
            o_ref[m] = (jnp.dot(h, WmT_ref[m], preferred_element_type=jnp.float32, precision=lax.Precision.HIGHEST)
                        + b_ref[2 + m][None, :])

    return pl.pallas_call(
        body,
        grid=(N // _R,),
        in_specs=[
            pl.BlockSpec((_R, D), lambda i: (i, 0)),
            pl.BlockSpec((2, _R, D), lambda i: (0, i, 0)),
            pl.BlockSpec((2, D, D), lambda i: (0, 0, 0)),
            pl.BlockSpec((2, D, D), lambda i: (0, 0, 0)),
            pl.BlockSpec((2, D, D), lambda i: (0, 0, 0)),
            pl.BlockSpec((8, D), lambda i: (0, 0)),
        ],
        out_specs=pl.BlockSpec((2, _R, D), lambda i: (0, i, 0)),
        out_shape=jax.ShapeDtypeStruct((2, N, D), jnp.float32),
    )(x, agg, A, B, WmT, biases)


def _tc_final(h1, agg, A, B, WmT, WoutT, Wqkvo, Wp1T, biases, biases2):
    H, HD = 4, 32
    scale = 1.0 / (HD ** 0.5)

    def body(h1_ref, agg_ref, A_ref, B_ref, WmT_ref, WoutT_ref, Wq_ref,
             Wp1_ref, b_ref, b2_ref, o_ref):
        u = []
        for m in range(2):
            hm = h1_ref[m]
            hd = jnp.dot(agg_ref[m], A_ref[m], preferred_element_type=jnp.float32, precision=lax.Precision.HIGHEST)
            hd = hd + jnp.dot(hm, B_ref[m], preferred_element_type=jnp.float32, precision=lax.Precision.HIGHEST)
            hd = hd + b_ref[m][None, :] + hm
            h = jnp.maximum(hd, 0.0)
            h2 = (jnp.dot(h, WmT_ref[m], preferred_element_type=jnp.float32, precision=lax.Precision.HIGHEST)
                  + b_ref[2 + m][None, :])
            u.append(jnp.dot(h2, WoutT_ref[m], preferred_element_type=jnp.float32, precision=lax.Precision.HIGHEST)
                     + b_ref[4 + m][None, :])

        q = [jnp.dot(u[m], Wq_ref[0], preferred_element_type=jnp.float32, precision=lax.Precision.HIGHEST)
             + b2_ref[0][None, :] for m in range(2)]
        kk = [jnp.dot(u[m], Wq_ref[1], preferred_element_type=jnp.float32, precision=lax.Precision.HIGHEST)
              + b2_ref[1][None, :] for m in range(2)]
        v = [jnp.dot(u[m], Wq_ref[2], preferred_element_type=jnp.float32, precision=lax.Precision.HIGHEST)
             + b2_ref[2][None, :] for m in range(2)]

        def head_scores(a, b):
            p = a * b
            return [jnp.sum(p[:, h * HD:(h + 1) * HD], axis=1, keepdims=True) * scale
                    for h in range(H)]

        s00 = head_scores(q[0], kk[0])
        s01 = head_scores(q[0], kk[1])
        s10 = head_scores(q[1], kk[0])
        s11 = head_scores(q[1], kk[1])

        att = []
        for si0, si1 in ((s00, s01), (s10, s11)):
            chunks = []
            for h in range(H):
                mx = jnp.maximum(si0[h], si1[h])
                e0 = jnp.exp(si0[h] - mx)
                e1 = jnp.exp(si1[h] - mx)
                inv = 1.0 / (e0 + e1)
                a0 = e0 * inv
                a1 = e1 * inv
                chunks.append(a0 * v[0][:, h * HD:(h + 1) * HD]
                              + a1 * v[1][:, h * HD:(h + 1) * HD])
            ctx = jnp.concatenate(chunks, axis=1)
            att.append(jnp.dot(ctx, Wq_ref[3], preferred_element_type=jnp.float32, precision=lax.Precision.HIGHEST)
                       + b2_ref[3][None, :])

        pooled = 0.5 * (att[0] + att[1])
        z = jnp.maximum(
            jnp.dot(pooled, Wp1_ref[...], preferred_element_type=jnp.float32, precision=lax.Precision.HIGHEST)
            + b2_ref[4][None, :], 0.0)
        o_ref[0, 0] = jnp.sum(z * b2_ref[5][None, :], axis=1) + b2_ref[6, 0]

    return pl.pallas_call(
        body,
        grid=(N // _R,),
        in_specs=[
            pl.BlockSpec((2, _R, D), lambda i: (0, i, 0)),
            pl.BlockSpec((2, _R, D), lambda i: (0, i, 0)),
            pl.BlockSpec((2, D, D), lambda i: (0, 0, 0)),
            pl.BlockSpec((2, D, D), lambda i: (0, 0, 0)),
            pl.BlockSpec((2, D, D), lambda i: (0, 0, 0)),
            pl.BlockSpec((2, D, D), lambda i: (0, 0, 0)),
            pl.BlockSpec((4, D, D), lambda i: (0, 0, 0)),
            pl.BlockSpec((D, D), lambda i: (0, 0)),
            pl.BlockSpec((8, D), lambda i: (0, 0)),
            pl.BlockSpec((8, D), lambda i: (0, 0)),
        ],
        out_specs=pl.BlockSpec((1, 1, _R), lambda i: (i, 0, 0)),
        out_shape=jax.ShapeDtypeStruct((N // _R, 1, _R), jnp.float32),
    )(h1, agg, A, B, WmT, WoutT, Wqkvo, Wp1T, biases, biases2)


def kernel(x, ei0, ei1, ei2, ei3, Wl, bl, W0, b0, W1, b1, Wm, bm, Wout, bout,
           Wq, bq, Wk, bk, Wv, bv, Wo, bo, Wp1, bp1, Wp2, bp2, mp_weights):
    def shape_idx(a):
        return a.reshape(NSUB, NCHUNK, CHUNK)

    g1 = jnp.stack([shape_idx(ei1[1]), shape_idx(ei3[1])])
    s1 = jnp.stack([shape_idx(ei1[0]), shape_idx(ei3[0])])
    g0 = jnp.stack([shape_idx(ei0[1]), shape_idx(ei2[1] + N)])
    s0 = jnp.stack([shape_idx(ei0[0]), shape_idx(ei2[0])])

    WlT = jnp.swapaxes(Wl, -1, -2)
    BT = jnp.swapaxes(W0 + W1, -1, -2)
    bsum = bl + b0 + b1
    WmT = jnp.swapaxes(Wm, -1, -2)
    WoutT = jnp.swapaxes(Wout, -1, -2) * mp_weights[:, None, None]
    boutw = bout * mp_weights[:, None]

    pad = jnp.zeros((4, D), jnp.float32)
    biases1 = jnp.concatenate([bsum[:, 1], bm[:, 1], pad], axis=0)
    biases0 = jnp.concatenate([bsum[:, 0], bm[:, 0], boutw,
                               jnp.zeros((2, D), jnp.float32)], axis=0)
    Wqkvo = jnp.stack([Wq.T, Wk.T, Wv.T, Wo.T])
    biases2 = jnp.stack([bq, bk, bv, bo, bp1, Wp2[0],
                         jnp.full((D,), bp2[0], jnp.float32),
                         jnp.zeros((D,), jnp.float32)])

    agg1 = _sc_propagate(x, g1, s1)
    h1 = _tc_layer1(x, agg1, WlT[:, 1], BT[:, 1], WmT[:, 1], biases1)

    agg0 = _sc_propagate(h1.reshape(2 * N, D), g0, s0)
    out = _tc_final(h1, agg0, WlT[:, 0], BT[:, 0], WmT[:, 0], WoutT,
                    Wqkvo, Wp1.T, biases0, biases2)
    return out.reshape(N)


# speedup vs baseline: 9.2993x; 1.2048x over previous
import functools

import jax
import jax.numpy as jnp
from jax import lax
from jax.experimental import pallas as pl
from jax.experimental.pallas import tpu as pltpu
from jax.experimental.pallas import tpu_sc as plsc

N = 10000
E = 320000
D = 128
NSUB = 16
NCORE = 2
CHUNK = 125
NCHUNK = E // (NSUB * CHUNK)
RPT = 624
ZC = 104


def _sc_propagate(src, gidx, sidx):
    mesh = plsc.VectorSubcoreMesh(core_axis_name="c", subcore_axis_name="s")

    @functools.partial(
        pl.kernel,
        mesh=mesh,
        out_type=jax.ShapeDtypeStruct((NCORE, N, D), jnp.float32),
        scratch_types=[
            pltpu.VMEM((NCHUNK // 4, CHUNK), jnp.int32),
            pltpu.VMEM((NCHUNK // 4, CHUNK), jnp.int32),
            pltpu.VMEM((CHUNK, D), jnp.float32),
            pltpu.VMEM((CHUNK, D), jnp.float32),
            pltpu.VMEM_SHARED((N, D), jnp.float32),
            pltpu.SemaphoreType.DMA,
            pltpu.SemaphoreType.DMA,
        ],
    )
    def k(src_hbm, gidx_hbm, sidx_hbm, out_hbm, gidx_v, sidx_v, rows0, rows1,
          acc, sem0, sem1):
        rows_v = rows0
        cid = lax.axis_index("c")
        sid = lax.axis_index("s")

        zeros16 = jnp.zeros((16,), jnp.float32)

        def zero_row(i, carry):
            for kk in range(D // 16):
                rows_v[i, pl.ds(kk * 16, 16)] = zeros16
            return carry

        lax.fori_loop(0, CHUNK, zero_row, 0)
        for b in range(RPT // ZC):
            pltpu.sync_copy(
                rows_v.at[pl.ds(0, ZC)], acc.at[pl.ds(sid * RPT + b * ZC, ZC)])

        @pl.when(sid == NSUB - 1)
        def _zero_tail():
            pltpu.sync_copy(rows_v.at[pl.ds(0, 16)],
                            acc.at[pl.ds(NSUB * RPT, 16)])

        plsc.subcore_barrier()

        Q = NCHUNK // 4

        def step(j, rows, sem):
            pltpu.make_async_copy(src_hbm.at[gidx_v.at[j]], rows, sem).wait()
            pltpu.sync_copy(rows, acc.at[sidx_v.at[j]], add=True)

        def body(p, carry):
            j = 2 * p
            step(j, rows0, sem0)
            pltpu.async_copy(src_hbm.at[gidx_v.at[j + 2]], rows0, sem0)
            step(j + 1, rows1, sem1)
            pltpu.async_copy(src_hbm.at[gidx_v.at[j + 3]], rows1, sem1)
            return carry

        for quarter in range(4):
            pltpu.sync_copy(gidx_hbm.at[cid, sid, pl.ds(quarter * Q, Q)],
                            gidx_v)
            pltpu.sync_copy(sidx_hbm.at[cid, sid, pl.ds(quarter * Q, Q)],
                            sidx_v)
            pltpu.async_copy(src_hbm.at[gidx_v.at[0]], rows0, sem0)
            pltpu.async_copy(src_hbm.at[gidx_v.at[1]], rows1, sem1)
            lax.fori_loop(0, Q // 2 - 1, body, 0)
            step(Q - 2, rows0, sem0)
            step(Q - 1, rows1, sem1)
        plsc.subcore_barrier()

        pltpu.sync_copy(
            acc.at[pl.ds(sid * RPT, RPT)],
            out_hbm.at[cid, pl.ds(sid * RPT, RPT)])

        @pl.when(sid == NSUB - 1)
        def _copy_tail():
            pltpu.sync_copy(acc.at[pl.ds(NSUB * RPT, 16)],
                            out_hbm.at[cid, pl.ds(NSUB * RPT, 16)])

    return k(src, gidx, sidx)


_R = 1000


def _tc_layer1(x, agg, A, B, WmT, biases):
    def body(x_ref, agg_ref, A_ref, B_ref, WmT_ref, b_ref, o_ref):
        xb = x_ref[...]
        for m in range(2):
            hd = jnp.dot(agg_ref[m], A_ref[m], preferred_element_type=jnp.float32, precision=lax.Precision.HIGHEST)
            hd = hd + jnp.dot(xb, B_ref[m], preferred_element_type=jnp.float32, precision=lax.Precision.HIGHEST)
            hd = hd + b_ref[m][None, :] + xb
            h = jnp.maximum(hd, 0.0)
            o_ref[m] = (jnp.dot(h, WmT_ref[m], preferred_element_type=jnp.float32, precision=lax.Precision.HIGHEST)
                        + b_ref[2 + m][None, :])

    return pl.pallas_call(
        body,
        grid=(N // _R,),
        in_specs=[
            pl.BlockSpec((_R, D), lambda i: (i, 0)),
            pl.BlockSpec((2, _R, D), lambda i: (0, i, 0)),
            pl.BlockSpec((2, D, D), lambda i: (0, 0, 0)),
            pl.BlockSpec((2, D, D), lambda i: (0, 0, 0)),
            pl.BlockSpec((2, D, D), lambda i: (0, 0, 0)),
            pl.BlockSpec((8, D), lambda i: (0, 0)),
        ],
        out_specs=pl.BlockSpec((2, _R, D), lambda i: (0, i, 0)),
        out_shape=jax.ShapeDtypeStruct((2, N, D), jnp.float32),
    )(x, agg, A, B, WmT, biases)


def _tc_final(h1, agg, A, B, Wfold, Wc, Mh, MhT, bpack):
    def body(h1_ref, agg_ref, A_ref, B_ref, Wf_ref, Wc_ref, Mh_ref, MhT_ref,
             b_ref, o_ref):
        qkv = []
        for m in range(2):
            hm = h1_ref[m]
            hd = jnp.dot(agg_ref[m], A_ref[m], preferred_element_type=jnp.float32, precision=lax.Precision.HIGHEST)
            hd = hd + jnp.dot(hm, B_ref[m], preferred_element_type=jnp.float32, precision=lax.Precision.HIGHEST)
            hd = hd + b_ref[m][None, :] + hm
            h = jnp.maximum(hd, 0.0)
            bqkv = jnp.concatenate(
                [b_ref[2 + 3 * m], b_ref[3 + 3 * m], b_ref[4 + 3 * m]])
            qkv.append(jnp.dot(h, Wf_ref[m], preferred_element_type=jnp.float32, precision=lax.Precision.HIGHEST)
                       + bqkv[None, :])

        q0, k0, v0 = qkv[0][:, :D], qkv[0][:, D:2 * D], qkv[0][:, 2 * D:]
        q1, k1, v1 = qkv[1][:, :D], qkv[1][:, D:2 * D], qkv[1][:, 2 * D:]

        pcat = jnp.concatenate([q0 * k0, q0 * k1, q1 * k0, q1 * k1], axis=1)
        S = jnp.dot(pcat, Mh_ref[...], preferred_element_type=jnp.float32)
        s00, s01 = S[:, 0:4], S[:, 4:8]
        s10, s11 = S[:, 8:12], S[:, 12:16]

        mx0 = jnp.maximum(s00, s01)
        mx1 = jnp.maximum(s10, s11)
        e00, e01 = jnp.exp(s00 - mx0), jnp.exp(s01 - mx0)
        e10, e11 = jnp.exp(s10 - mx1), jnp.exp(s11 - mx1)
        i0 = 1.0 / (e00 + e01)
        i1 = 1.0 / (e10 + e11)
        acat = jnp.concatenate(
            [e00 * i0, e01 * i0, e10 * i1, e11 * i1], axis=1)

        C = jnp.dot(acat, MhT_ref[...], preferred_element_type=jnp.float32, precision=lax.Precision.HIGHEST)
        ctxsum = C[:, :D] * v0 + C[:, D:] * v1

        z = jnp.maximum(
            jnp.dot(ctxsum, Wc_ref[...], preferred_element_type=jnp.float32, precision=lax.Precision.HIGHEST)
            + b_ref[8][None, :], 0.0)
        o_ref[0, 0] = jnp.sum(z * b_ref[9][None, :], axis=1) + b_ref[10, 0]

    return pl.pallas_call(
        body,
        grid=(N // _R,),
        in_specs=[
            pl.BlockSpec((2, _R, D), lambda i: (0, i, 0)),
            pl.BlockSpec((2, _R, D), lambda i: (0, i, 0)),
            pl.BlockSpec((2, D, D), lambda i: (0, 0, 0)),
            pl.BlockSpec((2, D, D), lambda i: (0, 0, 0)),
            pl.BlockSpec((2, D, 3 * D), lambda i: (0, 0, 0)),
            pl.BlockSpec((D, D), lambda i: (0, 0)),
            pl.BlockSpec((4 * D, D), lambda i: (0, 0)),
            pl.BlockSpec((16, 2 * D), lambda i: (0, 0)),
            pl.BlockSpec((16, D), lambda i: (0, 0)),
        ],
        out_specs=pl.BlockSpec((1, 1, _R), lambda i: (i, 0, 0)),
        out_shape=jax.ShapeDtypeStruct((N // _R, 1, _R), jnp.float32),
    )(h1, agg, A, B, Wfold, Wc, Mh, MhT, bpack)


def kernel(x, ei0, ei1, ei2, ei3, Wl, bl, W0, b0, W1, b1, Wm, bm, Wout, bout,
           Wq, bq, Wk, bk, Wv, bv, Wo, bo, Wp1, bp1, Wp2, bp2, mp_weights):
    def shape_idx(a):
        return a.reshape(NSUB, NCHUNK, CHUNK)

    g1 = jnp.stack([shape_idx(ei1[1]), shape_idx(ei3[1])])
    s1 = jnp.stack([shape_idx(ei1[0]), shape_idx(ei3[0])])
    g0 = jnp.stack([shape_idx(ei0[1]), shape_idx(ei2[1] + N)])
    s0 = jnp.stack([shape_idx(ei0[0]), shape_idx(ei2[0])])

    WlT = jnp.swapaxes(Wl, -1, -2)
    BT = jnp.swapaxes(W0 + W1, -1, -2)
    bsum = bl + b0 + b1
    WmT = jnp.swapaxes(Wm, -1, -2)
    WoutT = jnp.swapaxes(Wout, -1, -2) * mp_weights[:, None, None]
    boutw = bout * mp_weights[:, None]

    pad = jnp.zeros((4, D), jnp.float32)
    biases1 = jnp.concatenate([bsum[:, 1], bm[:, 1], pad], axis=0)

    Wcat = jnp.stack([
        jnp.concatenate([WoutT[m] @ Wq.T, WoutT[m] @ Wk.T, WoutT[m] @ Wv.T],
                        axis=1) for m in range(2)])
    bcat = jnp.stack([
        jnp.concatenate([boutw[m] @ Wq.T + bq, boutw[m] @ Wk.T + bk,
                         boutw[m] @ Wv.T + bv]) for m in range(2)])
    Wfold = jnp.stack([WmT[m, 0] @ Wcat[m] for m in range(2)])
    bfold = jnp.stack([bm[m, 0] @ Wcat[m] + bcat[m] for m in range(2)])

    scale = 1.0 / jnp.sqrt(jnp.float32(D // 4))
    Hmat = (jnp.arange(D)[:, None] // (D // 4)
            == jnp.arange(4)[None, :]).astype(jnp.float32)
    Mh = jnp.concatenate(
        [jnp.kron(jnp.eye(4, dtype=jnp.float32), Hmat) * scale,
         jnp.zeros((4 * D, D - 16), jnp.float32)], axis=1)
    P = jnp.array([[1, 0], [0, 1], [1, 0], [0, 1]], jnp.float32)
    MhT = jnp.kron(P, Hmat.T)

    Wc = 0.5 * (Wo.T @ Wp1.T)
    bc = bo @ Wp1.T + bp1
    bpack = jnp.concatenate([
        bsum[:, 0],
        bfold.reshape(6, D),
        bc[None, :], Wp2, jnp.full((1, D), bp2[0]),
        jnp.zeros((5, D), jnp.float32)], axis=0)

    agg1 = _sc_propagate(x, g1, s1)
    h1 = _tc_layer1(x, agg1, WlT[:, 1], BT[:, 1], WmT[:, 1], biases1)

    agg0 = _sc_propagate(h1.reshape(2 * N, D), g0, s0)
    out = _tc_final(h1, agg0, WlT[:, 0], BT[:, 0], Wfold, Wc, Mh, MhT, bpack)
    return out.reshape(N)


# speedup vs baseline: 9.8495x; 1.0592x over previous
import functools

import jax
import jax.numpy as jnp
from jax import lax
from jax.experimental import pallas as pl
from jax.experimental.pallas import tpu as pltpu
from jax.experimental.pallas import tpu_sc as plsc

N = 10000
E = 320000
D = 128
NSUB = 16
NCORE = 2
CHUNK = 125
NCHUNK = E // (NSUB * CHUNK)
RPT = 624
ZC = 104


def _sc_propagate(src, gidx, sidx):
    mesh = plsc.VectorSubcoreMesh(core_axis_name="c", subcore_axis_name="s")

    @functools.partial(
        pl.kernel,
        mesh=mesh,
        out_type=jax.ShapeDtypeStruct((NCORE, N, D), jnp.float32),
        scratch_types=[
            pltpu.VMEM((NCHUNK // 4, CHUNK), jnp.int32),
            pltpu.VMEM((NCHUNK // 4, CHUNK), jnp.int32),
            pltpu.VMEM((CHUNK, D), jnp.float32),
            pltpu.VMEM((CHUNK, D), jnp.float32),
            pltpu.VMEM_SHARED((N, D), jnp.float32),
            pltpu.SemaphoreType.DMA,
            pltpu.SemaphoreType.DMA,
        ],
    )
    def k(src_hbm, gidx_hbm, sidx_hbm, out_hbm, gidx_v, sidx_v, rows0, rows1,
          acc, sem0, sem1):
        rows_v = rows0
        cid = lax.axis_index("c")
        sid = lax.axis_index("s")

        zeros16 = jnp.zeros((16,), jnp.float32)

        def zero_row(i, carry):
            for kk in range(D // 16):
                rows_v[i, pl.ds(kk * 16, 16)] = zeros16
            return carry

        lax.fori_loop(0, CHUNK, zero_row, 0)
        for b in range(RPT // ZC):
            pltpu.sync_copy(
                rows_v.at[pl.ds(0, ZC)], acc.at[pl.ds(sid * RPT + b * ZC, ZC)])

        @pl.when(sid == NSUB - 1)
        def _zero_tail():
            pltpu.sync_copy(rows_v.at[pl.ds(0, 16)],
                            acc.at[pl.ds(NSUB * RPT, 16)])

        plsc.subcore_barrier()

        Q = NCHUNK // 4

        def step(j, rows, sem):
            pltpu.make_async_copy(src_hbm.at[gidx_v.at[j]], rows, sem).wait()
            pltpu.sync_copy(rows, acc.at[sidx_v.at[j]], add=True)

        def body(p, carry):
            j = 2 * p
            step(j, rows0, sem0)
            pltpu.async_copy(src_hbm.at[gidx_v.at[j + 2]], rows0, sem0)
            step(j + 1, rows1, sem1)
            pltpu.async_copy(src_hbm.at[gidx_v.at[j + 3]], rows1, sem1)
            return carry

        for quarter in range(4):
            pltpu.sync_copy(gidx_hbm.at[cid, sid, pl.ds(quarter * Q, Q)],
                            gidx_v)
            pltpu.sync_copy(sidx_hbm.at[cid, sid, pl.ds(quarter * Q, Q)],
                            sidx_v)
            pltpu.async_copy(src_hbm.at[gidx_v.at[0]], rows0, sem0)
            pltpu.async_copy(src_hbm.at[gidx_v.at[1]], rows1, sem1)
            lax.fori_loop(0, Q // 2 - 1, body, 0)
            step(Q - 2, rows0, sem0)
            step(Q - 1, rows1, sem1)
        plsc.subcore_barrier()

        pltpu.sync_copy(
            acc.at[pl.ds(sid * RPT, RPT)],
            out_hbm.at[cid, pl.ds(sid * RPT, RPT)])

        @pl.when(sid == NSUB - 1)
        def _copy_tail():
            pltpu.sync_copy(acc.at[pl.ds(NSUB * RPT, 16)],
                            out_hbm.at[cid, pl.ds(NSUB * RPT, 16)])

    return k(src, gidx, sidx)


_R = 2000


def _tc_layer1(x, agg, A, B, WmT, biases):
    def body(x_ref, agg_ref, A_ref, B_ref, WmT_ref, b_ref, o_ref):
        xb = x_ref[...]
        for m in range(2):
            hd = jnp.dot(agg_ref[m], A_ref[m], preferred_element_type=jnp.float32, precision=lax.Precision.HIGHEST)
            hd = hd + jnp.dot(xb, B_ref[m], preferred_element_type=jnp.float32, precision=lax.Precision.HIGHEST)
            hd = hd + b_ref[m][None, :] + xb
            h = jnp.maximum(hd, 0.0)
            o_ref[m] = (jnp.dot(h, WmT_ref[m], preferred_element_type=jnp.float32, precision=lax.Precision.HIGHEST)
                        + b_ref[2 + m][None, :])

    return pl.pallas_call(
        body,
        grid=(N // _R,),
        in_specs=[
            pl.BlockSpec((_R, D), lambda i: (i, 0)),
            pl.BlockSpec((2, _R, D), lambda i: (0, i, 0)),
            pl.BlockSpec((2, D, D), lambda i: (0, 0, 0)),
            pl.BlockSpec((2, D, D), lambda i: (0, 0, 0)),
            pl.BlockSpec((2, D, D), lambda i: (0, 0, 0)),
            pl.BlockSpec((8, D), lambda i: (0, 0)),
        ],
        out_specs=pl.BlockSpec((2, _R, D), lambda i: (0, i, 0)),
        out_shape=jax.ShapeDtypeStruct((2, N, D), jnp.float32),
    )(x, agg, A, B, WmT, biases)


def _tc_final(h1, agg, A, B, Wfold, Wc, Mh, MhT, bpack):
    def body(h1_ref, agg_ref, A_ref, B_ref, Wf_ref, Wc_ref, Mh_ref, MhT_ref,
             b_ref, o_ref):
        qkv = []
        for m in range(2):
            hm = h1_ref[m]
            hd = jnp.dot(agg_ref[m], A_ref[m], preferred_element_type=jnp.float32, precision=lax.Precision.HIGHEST)
            hd = hd + jnp.dot(hm, B_ref[m], preferred_element_type=jnp.float32, precision=lax.Precision.HIGHEST)
            hd = hd + b_ref[m][None, :] + hm
            h = jnp.maximum(hd, 0.0)
            bqkv = jnp.concatenate(
                [b_ref[2 + 3 * m], b_ref[3 + 3 * m], b_ref[4 + 3 * m]])
            qkv.append(jnp.dot(h, Wf_ref[m], preferred_element_type=jnp.float32, precision=lax.Precision.HIGHEST)
                       + bqkv[None, :])

        q0, k0, v0 = qkv[0][:, :D], qkv[0][:, D:2 * D], qkv[0][:, 2 * D:]
        q1, k1, v1 = qkv[1][:, :D], qkv[1][:, D:2 * D], qkv[1][:, 2 * D:]

        pcat = jnp.concatenate([q0 * k0, q0 * k1, q1 * k0, q1 * k1], axis=1)
        S = jnp.dot(pcat, Mh_ref[...], preferred_element_type=jnp.float32)
        s00, s01 = S[:, 0:4], S[:, 4:8]
        s10, s11 = S[:, 8:12], S[:, 12:16]

        mx0 = jnp.maximum(s00, s01)
        mx1 = jnp.maximum(s10, s11)
        e00, e01 = jnp.exp(s00 - mx0), jnp.exp(s01 - mx0)
        e10, e11 = jnp.exp(s10 - mx1), jnp.exp(s11 - mx1)
        i0 = 1.0 / (e00 + e01)
        i1 = 1.0 / (e10 + e11)
        acat = jnp.concatenate(
            [e00 * i0, e01 * i0, e10 * i1, e11 * i1], axis=1)

        C = jnp.dot(acat, MhT_ref[...], preferred_element_type=jnp.float32, precision=lax.Precision.HIGHEST)
        ctxsum = C[:, :D] * v0 + C[:, D:] * v1

        z = jnp.maximum(
            jnp.dot(ctxsum, Wc_ref[...], preferred_element_type=jnp.float32, precision=lax.Precision.HIGHEST)
            + b_ref[8][None, :], 0.0)
        o_ref[0, 0] = jnp.sum(z * b_ref[9][None, :], axis=1) + b_ref[10, 0]

    return pl.pallas_call(
        body,
        grid=(N // _R,),
        in_specs=[
            pl.BlockSpec((2, _R, D), lambda i: (0, i, 0)),
            pl.BlockSpec((2, _R, D), lambda i: (0, i, 0)),
            pl.BlockSpec((2, D, D), lambda i: (0, 0, 0)),
            pl.BlockSpec((2, D, D), lambda i: (0, 0, 0)),
            pl.BlockSpec((2, D, 3 * D), lambda i: (0, 0, 0)),
            pl.BlockSpec((D, D), lambda i: (0, 0)),
            pl.BlockSpec((4 * D, D), lambda i: (0, 0)),
            pl.BlockSpec((16, 2 * D), lambda i: (0, 0)),
            pl.BlockSpec((16, D), lambda i: (0, 0)),
        ],
        out_specs=pl.BlockSpec((1, 1, _R), lambda i: (i, 0, 0)),
        out_shape=jax.ShapeDtypeStruct((N // _R, 1, _R), jnp.float32),
    )(h1, agg, A, B, Wfold, Wc, Mh, MhT, bpack)


def kernel(x, ei0, ei1, ei2, ei3, Wl, bl, W0, b0, W1, b1, Wm, bm, Wout, bout,
           Wq, bq, Wk, bk, Wv, bv, Wo, bo, Wp1, bp1, Wp2, bp2, mp_weights):
    def shape_idx(a):
        return a.reshape(NSUB, NCHUNK, CHUNK)

    g1 = jnp.stack([shape_idx(ei1[1]), shape_idx(ei3[1])])
    s1 = jnp.stack([shape_idx(ei1[0]), shape_idx(ei3[0])])
    g0 = jnp.stack([shape_idx(ei0[1]), shape_idx(ei2[1] + N)])
    s0 = jnp.stack([shape_idx(ei0[0]), shape_idx(ei2[0])])

    WlT = jnp.swapaxes(Wl, -1, -2)
    BT = jnp.swapaxes(W0 + W1, -1, -2)
    bsum = bl + b0 + b1
    WmT = jnp.swapaxes(Wm, -1, -2)
    WoutT = jnp.swapaxes(Wout, -1, -2) * mp_weights[:, None, None]
    boutw = bout * mp_weights[:, None]

    pad = jnp.zeros((4, D), jnp.float32)
    biases1 = jnp.concatenate([bsum[:, 1], bm[:, 1], pad], axis=0)

    Wcat = jnp.stack([
        jnp.concatenate([WoutT[m] @ Wq.T, WoutT[m] @ Wk.T, WoutT[m] @ Wv.T],
                        axis=1) for m in range(2)])
    bcat = jnp.stack([
        jnp.concatenate([boutw[m] @ Wq.T + bq, boutw[m] @ Wk.T + bk,
                         boutw[m] @ Wv.T + bv]) for m in range(2)])
    Wfold = jnp.stack([WmT[m, 0] @ Wcat[m] for m in range(2)])
    bfold = jnp.stack([bm[m, 0] @ Wcat[m] + bcat[m] for m in range(2)])

    scale = 1.0 / jnp.sqrt(jnp.float32(D // 4))
    Hmat = (jnp.arange(D)[:, None] // (D // 4)
            == jnp.arange(4)[None, :]).astype(jnp.float32)
    Mh = jnp.concatenate(
        [jnp.kron(jnp.eye(4, dtype=jnp.float32), Hmat) * scale,
         jnp.zeros((4 * D, D - 16), jnp.float32)], axis=1)
    P = jnp.array([[1, 0], [0, 1], [1, 0], [0, 1]], jnp.float32)
    MhT = jnp.kron(P, Hmat.T)

    Wc = 0.5 * (Wo.T @ Wp1.T)
    bc = bo @ Wp1.T + bp1
    bpack = jnp.concatenate([
        bsum[:, 0],
        bfold.reshape(6, D),
        bc[None, :], Wp2, jnp.full((1, D), bp2[0]),
        jnp.zeros((5, D), jnp.float32)], axis=0)

    agg1 = _sc_propagate(x, g1, s1)
    h1 = _tc_layer1(x, agg1, WlT[:, 1], BT[:, 1], WmT[:, 1], biases1)

    agg0 = _sc_propagate(h1.reshape(2 * N, D), g0, s0)
    out = _tc_final(h1, agg0, WlT[:, 0], BT[:, 0], Wfold, Wc, Mh, MhT, bpack)
    return out.reshape(N)
